# baseline (speedup 1.0000x reference)
import functools

import jax
import jax.numpy as jnp
from jax.experimental import pallas as pl
from jax.experimental.pallas import tpu as pltpu

_N = 2048
_H = 12
_BLK = 256


def _argsort_kernel(x_row_ref, c_row_ref, xt_ref, ct_ref, out_ref, scol_ref):
    w_row = x_row_ref[0, 0:1, :]
    c_row = c_row_ref[0, 0:1, :]
    w_col = xt_ref[0, :, 0:1]
    c_col = ct_ref[0, :, 0:1]
    for h in range(1, _H):
        w_row = w_row + x_row_ref[0, h:h + 1, :]
        c_row = c_row + c_row_ref[0, h:h + 1, :]
        w_col = w_col + xt_ref[0, :, h:h + 1]
        c_col = c_col + ct_ref[0, :, h:h + 1]
    s_row = (w_row * c_row) * (1.0 / (_H * _H))
    scol_ref[...] = (w_col * c_col) * (1.0 / (_H * _H))

    j_row = jax.lax.broadcasted_iota(jnp.int32, (1, _N), 1)
    i_sub = jax.lax.broadcasted_iota(jnp.int32, (_BLK, 1), 0)

    def body(blk, acc):
        base = blk * _BLK
        sc = scol_ref[pl.ds(base, _BLK), :]
        i_col = i_sub + base
        gt = (s_row > sc).astype(jnp.int32)
        tie = ((s_row == sc) & (j_row < i_col)).astype(jnp.int32)
        rank = jnp.sum(gt + tie, axis=1, keepdims=True)
        acc = acc + jnp.sum(
            jnp.where(rank == j_row, i_col, 0), axis=0, keepdims=True)
        return acc

    acc0 = jnp.zeros((1, _N), jnp.int32)
    acc = jax.lax.fori_loop(0, _N // _BLK, body, acc0)
    out_ref[...] = acc.reshape(1, 1, _N)


@jax.jit
def kernel(x, contributions):
    b = x.shape[0]
    x_flat = x.reshape(b, _H, -1)
    xt = jnp.swapaxes(x[:, :, 0, :], 1, 2)
    ct = jnp.swapaxes(contributions, 1, 2)
    grid = (b,)
    return pl.pallas_call(
        _argsort_kernel,
        grid=grid,
        in_specs=[
            pl.BlockSpec((1, _H, _N), lambda i: (i, 0, 0)),
            pl.BlockSpec((1, _H, _N), lambda i: (i, 0, 0)),
            pl.BlockSpec((1, _N, _H), lambda i: (i, 0, 0)),
            pl.BlockSpec((1, _N, _H), lambda i: (i, 0, 0)),
        ],
        out_specs=pl.BlockSpec((1, 1, _N), lambda i: (i, 0, 0)),
        out_shape=jax.ShapeDtypeStruct((b, 1, _N), jnp.int32),
        scratch_shapes=[pltpu.VMEM((_N, 1), jnp.float32)],
    )(x_flat, contributions, xt, ct).reshape(b, _N)


# speedup vs baseline: 652.4323x; 652.4323x over previous
import jax
import jax.numpy as jnp
from jax.experimental import pallas as pl
from jax.experimental.pallas import tpu as pltpu

_N = 2048
_H = 12
_R = 16
_C = 128


def _sort_kernel(x_ref, c_ref, out_ref, key_scratch):
    w = x_ref[0, 0, 0:1, :]
    c = c_ref[0, 0:1, :]
    for h in range(1, _H):
        w = w + x_ref[0, h, 0:1, :]
        c = c + c_ref[0, h:h + 1, :]
    s = (c / float(_H)) * (w / float(_H))
    s = s + 0.0
    ib = jax.lax.bitcast_convert_type(s, jnp.int32)
    key_row = ib ^ ((ib >> 31) & jnp.int32(0x7FFFFFFF))

    for i in range(_R):
        key_scratch[i:i + 1, :] = key_row[:, _C * i:_C * (i + 1)]
    key = key_scratch[...]

    rows = jax.lax.broadcasted_iota(jnp.int32, (_R, _C), 0)
    cols = jax.lax.broadcasted_iota(jnp.int32, (_R, _C), 1)
    p = rows * _C + cols
    idx = p

    blk = 2
    while blk <= _N:
        asc = (p & blk) == 0
        d = blk // 2
        while d >= 1:
            lower = (p & d) == 0
            sel = lower == asc
            if d >= _C:
                e = d // _C
                pk = jnp.where(lower, jnp.roll(key, -e, axis=0),
                               jnp.roll(key, e, axis=0))
                pi = jnp.where(lower, jnp.roll(idx, -e, axis=0),
                               jnp.roll(idx, e, axis=0))
            else:
                pk = jnp.where(lower, jnp.roll(key, -d, axis=1),
                               jnp.roll(key, d, axis=1))
                pi = jnp.where(lower, jnp.roll(idx, -d, axis=1),
                               jnp.roll(idx, d, axis=1))
            v_first = (key > pk) | ((key == pk) & (idx < pi))
            keep_v = v_first == sel
            key = jnp.where(keep_v, key, pk)
            idx = jnp.where(keep_v, idx, pi)
            d //= 2
        blk *= 2

    out_ref[...] = idx.reshape(1, _R, _C)


@jax.jit
def kernel(x, contributions):
    b = x.shape[0]
    return pl.pallas_call(
        _sort_kernel,
        grid=(b,),
        in_specs=[
            pl.BlockSpec((1, _H, 8, _N), lambda i: (i, 0, 0, 0)),
            pl.BlockSpec((1, _H, _N), lambda i: (i, 0, 0)),
        ],
        out_specs=pl.BlockSpec((1, _R, _C), lambda i: (i, 0, 0)),
        out_shape=jax.ShapeDtypeStruct((b, _R, _C), jnp.int32),
        scratch_shapes=[pltpu.VMEM((_R, _C), jnp.int32)],
    )(x, contributions).reshape(b, _N)


# speedup vs baseline: 877.4249x; 1.3449x over previous
import jax
import jax.numpy as jnp
from jax.experimental import pallas as pl
from jax.experimental.pallas import tpu as pltpu

_N = 2048
_H = 12
_R = 32
_C = 128


def _sort_kernel(x_ref, c_ref, out_ref, key_scratch):
    for b in range(2):
        w = x_ref[b, 0, 0:1, :]
        c = c_ref[b, 0:1, :]
        for h in range(1, _H):
            w = w + x_ref[b, h, 0:1, :]
            c = c + c_ref[b, h:h + 1, :]
        s = (c / float(_H)) * (w / float(_H))
        s = s + 0.0
        ib = jax.lax.bitcast_convert_type(s, jnp.int32)
        key_row = ib ^ ((ib >> 31) & jnp.int32(0x7FFFFFFF))
        for i in range(16):
            key_scratch[b * 16 + i:b * 16 + i + 1, :] = \
                key_row[:, _C * i:_C * (i + 1)]
    key = key_scratch[...]

    rows = jax.lax.broadcasted_iota(jnp.int32, (_R, _C), 0)
    cols = jax.lax.broadcasted_iota(jnp.int32, (_R, _C), 1)
    p = (rows & 15) * _C + cols
    idx = p

    blk = 2
    while blk <= _N:
        asc = (p & blk) == 0
        d = blk // 2
        while d >= 1:
            lower = (p & d) == 0
            sel = lower == asc
            if d >= _C:
                e = d // _C
                pk = jnp.where(lower, jnp.roll(key, -e, axis=0),
                               jnp.roll(key, e, axis=0))
                pi = jnp.where(lower, jnp.roll(idx, -e, axis=0),
                               jnp.roll(idx, e, axis=0))
            else:
                pk = jnp.where(lower, jnp.roll(key, -d, axis=1),
                               jnp.roll(key, d, axis=1))
                pi = jnp.where(lower, jnp.roll(idx, -d, axis=1),
                               jnp.roll(idx, d, axis=1))
            v_first = (key > pk) | ((key == pk) & (idx < pi))
            keep_v = v_first == sel
            key = jnp.where(keep_v, key, pk)
            idx = jnp.where(keep_v, idx, pi)
            d //= 2
        blk *= 2

    out_ref[...] = idx.reshape(2, 16, _C)


@jax.jit
def kernel(x, contributions):
    b = x.shape[0]
    return pl.pallas_call(
        _sort_kernel,
        grid=(1,),
        in_specs=[
            pl.BlockSpec((b, _H, 8, _N), lambda i: (0, 0, 0, 0)),
            pl.BlockSpec((b, _H, _N), lambda i: (0, 0, 0)),
        ],
        out_specs=pl.BlockSpec((b, 16, _C), lambda i: (0, 0, 0)),
        out_shape=jax.ShapeDtypeStruct((b, 16, _C), jnp.int32),
        scratch_shapes=[pltpu.VMEM((_R, _C), jnp.int32)],
    )(x, contributions).reshape(b, _N)


# speedup vs baseline: 1035.4624x; 1.1801x over previous
import jax
import jax.numpy as jnp
from jax.experimental import pallas as pl
from jax.experimental.pallas import tpu as pltpu

_N = 2048
_H = 12
_R = 32
_C = 128


def _sort_kernel(x_ref, c_ref, out_ref, key_scratch):
    for b in range(2):
        w = x_ref[b, 0, 0:1, :]
        c = c_ref[b, 0:1, :]
        for h in range(1, _H):
            w = w + x_ref[b, h, 0:1, :]
            c = c + c_ref[b, h:h + 1, :]
        s = (c / float(_H)) * (w / float(_H))
        s = s + 0.0
        ib = jax.lax.bitcast_convert_type(s, jnp.int32)
        key_row = ib ^ ((ib >> 31) & jnp.int32(0x7FFFFFFF))
        for i in range(16):
            key_scratch[b * 16 + i:b * 16 + i + 1, :] = \
                key_row[:, _C * i:_C * (i + 1)]
    key = key_scratch[...]

    rows = jax.lax.broadcasted_iota(jnp.int32, (_R, _C), 0)
    cols = jax.lax.broadcasted_iota(jnp.int32, (_R, _C), 1)
    p = cols * 16 + (rows & 15)
    idx = (rows & 15) * _C + cols

    blk = 2
    while blk <= _N:
        asc = (p & blk) == 0
        d = blk // 2
        while d >= 1:
            lower = (p & d) == 0
            sel = lower == asc
            if d >= 16:
                e = d // 16
                pk = jnp.where(lower, jnp.roll(key, -e, axis=1),
                               jnp.roll(key, e, axis=1))
                pi = jnp.where(lower, jnp.roll(idx, -e, axis=1),
                               jnp.roll(idx, e, axis=1))
            else:
                pk = jnp.where(lower, jnp.roll(key, -d, axis=0),
                               jnp.roll(key, d, axis=0))
                pi = jnp.where(lower, jnp.roll(idx, -d, axis=0),
                               jnp.roll(idx, d, axis=0))
            v_first = (key > pk) | ((key == pk) & (idx < pi))
            keep_v = v_first == sel
            key = jnp.where(keep_v, key, pk)
            idx = jnp.where(keep_v, idx, pi)
            d //= 2
        blk *= 2

    t = jnp.swapaxes(idx, 0, 1)
    out_ref[0:1] = t[:, 0:16].reshape(1, _C, 16)
    out_ref[1:2] = t[:, 16:32].reshape(1, _C, 16)


@jax.jit
def kernel(x, contributions):
    b = x.shape[0]
    return pl.pallas_call(
        _sort_kernel,
        grid=(1,),
        in_specs=[
            pl.BlockSpec((b, _H, 8, _N), lambda i: (0, 0, 0, 0)),
            pl.BlockSpec((b, _H, _N), lambda i: (0, 0, 0)),
        ],
        out_specs=pl.BlockSpec((b, _C, 16), lambda i: (0, 0, 0)),
        out_shape=jax.ShapeDtypeStruct((b, _C, 16), jnp.int32),
        scratch_shapes=[pltpu.VMEM((_R, _C), jnp.int32)],
    )(x, contributions).reshape(b, _N)
